# baseline (speedup 1.0000x reference)
import functools

import jax
import jax.numpy as jnp
from jax import lax
from jax.experimental import pallas as pl
from jax.experimental.pallas import tpu as pltpu
from jax.experimental.pallas import tpu_sc as plsc

B = 4096
K = 200
D = 32
NN = 10
W1 = 1e-07
W2 = 1.0
W3 = 1e-07
W4 = 1.0
NEG_WEIGHT = 200.0
GAMMA = 1e-04
LM = 2.75

NC = 2
NS = 16
L = 16
NW = NC * NS
BPW = B // NW
CH = 128

NEG_CHUNKS = BPW * K // CH
NBH_CHUNKS = BPW * NN // CH
TBL_ROWS = 250000
TC_GRID = 50
TBL_BLK = TBL_ROWS // TC_GRID
NS_BLK = B * K // TC_GRID


def _iota16():
    return lax.iota(jnp.int32, L)


def _dots16(rows_ref, u_ref, jv, ev):
    acc = jnp.zeros((L,), jnp.float32)
    for d in range(D):
        dv = jnp.full((L,), d, jnp.int32)
        acc = acc + plsc.load_gather(rows_ref, [jv, dv]) * plsc.load_gather(u_ref, [ev, dv])
    return acc


def _sc_body(users, pos, negf, Gu, Gi, bu, bi, nmatf, cmatf,
             pos_s_o, pos_w_o, neg_s_o, neg_w_o, sc_s_o, sim_o,
             uidx_v, pidx_v, urows_v, prows_v, buv, bipv,
             cidx_v, nvals_v, grows_v, betan_v, score_v, wv_v, simc_v, sem):
    wid = lax.axis_index("s") * NC + lax.axis_index("c")
    base = pl.multiple_of(wid * BPW, BPW)

    pltpu.sync_copy(users.at[pl.ds(base, BPW)], uidx_v)
    pltpu.sync_copy(pos.at[pl.ds(base, BPW)], pidx_v)

    c1 = pltpu.async_copy(Gu.at[uidx_v], urows_v, sem)
    c2 = pltpu.async_copy(Gi.at[pidx_v], prows_v, sem)
    c3 = pltpu.async_copy(bu.at[uidx_v], buv, sem)
    c4 = pltpu.async_copy(bi.at[pidx_v], bipv, sem)
    c1.wait()
    c2.wait()
    c3.wait()
    c4.wait()

    def pos_group(g, _):
        jv = g * L + _iota16()
        acc = _dots16(prows_v, urows_v, jv, jv)
        score_v[pl.ds(g * L, L)] = acc
        w = W1 + W2 * buv[pl.ds(g * L, L)] * bipv[pl.ds(g * L, L)]
        wv_v[pl.ds(g * L, L)] = w
        return 0
    lax.fori_loop(0, BPW // L, pos_group, 0)
    pltpu.sync_copy(score_v, pos_s_o.at[pl.ds(base, BPW)])
    pltpu.sync_copy(wv_v, pos_w_o.at[pl.ds(base, BPW)])

    def nbh_chunk(t, _):
        jbase = t * CH

        def build_idx(g, _):
            jv = jbase + g * L + _iota16()
            ev = lax.div(jv, NN)
            rv = lax.rem(jv, NN)
            pid = plsc.load_gather(pidx_v, [ev])
            cidx_v[pl.ds(g * L, L)] = pid * NN + rv
            return 0
        lax.fori_loop(0, CH // L, build_idx, 0)

        g1 = pltpu.async_copy(nmatf.at[cidx_v], nvals_v, sem)
        g2 = pltpu.async_copy(cmatf.at[cidx_v], simc_v, sem)
        g1.wait()
        g2.wait()
        g3 = pltpu.async_copy(Gi.at[nvals_v], grows_v, sem)
        g3.wait()

        def dot_group(g, _):
            jloc = g * L + _iota16()
            ev = lax.div(jbase + jloc, NN)
            acc = _dots16(grows_v, urows_v, jloc, ev)
            score_v[pl.ds(g * L, L)] = acc
            return 0
        lax.fori_loop(0, CH // L, dot_group, 0)

        off = pl.multiple_of(base * NN + jbase, 8)
        pltpu.sync_copy(score_v, sc_s_o.at[pl.ds(off, CH)])
        pltpu.sync_copy(simc_v, sim_o.at[pl.ds(off, CH)])
        return 0
    lax.fori_loop(0, NBH_CHUNKS, nbh_chunk, 0)

    def neg_chunk(t, _):
        jbase = t * CH
        off = pl.multiple_of(base * K + jbase, 8)
        pltpu.sync_copy(negf.at[pl.ds(off, CH)], cidx_v)
        g1 = pltpu.async_copy(Gi.at[cidx_v], grows_v, sem)
        g2 = pltpu.async_copy(bi.at[cidx_v], betan_v, sem)
        g1.wait()
        g2.wait()

        def dot_group(g, _):
            jloc = g * L + _iota16()
            ev = lax.div(jbase + jloc, K)
            acc = _dots16(grows_v, urows_v, jloc, ev)
            score_v[pl.ds(g * L, L)] = acc
            bug = plsc.load_gather(buv, [ev])
            w = W3 + W4 * bug * betan_v[pl.ds(g * L, L)]
            wv_v[pl.ds(g * L, L)] = w
            return 0
        lax.fori_loop(0, CH // L, dot_group, 0)

        pltpu.sync_copy(score_v, neg_s_o.at[pl.ds(off, CH)])
        pltpu.sync_copy(wv_v, neg_w_o.at[pl.ds(off, CH)])
        return 0
    lax.fori_loop(0, NEG_CHUNKS, neg_chunk, 0)


@jax.jit
def _sc_call(users, pos, negf, Gu, Gi, bu, bi, nmatf, cmatf):
    mesh = plsc.VectorSubcoreMesh(core_axis_name="c", subcore_axis_name="s")
    f32 = jnp.float32
    out_type = (
        jax.ShapeDtypeStruct((B,), f32),
        jax.ShapeDtypeStruct((B,), f32),
        jax.ShapeDtypeStruct((B * K,), f32),
        jax.ShapeDtypeStruct((B * K,), f32),
        jax.ShapeDtypeStruct((B * NN,), f32),
        jax.ShapeDtypeStruct((B * NN,), f32),
    )
    scratch = [
        pltpu.VMEM((BPW,), jnp.int32),
        pltpu.VMEM((BPW,), jnp.int32),
        pltpu.VMEM((BPW, D), f32),
        pltpu.VMEM((BPW, D), f32),
        pltpu.VMEM((BPW,), f32),
        pltpu.VMEM((BPW,), f32),
        pltpu.VMEM((CH,), jnp.int32),
        pltpu.VMEM((CH,), jnp.int32),
        pltpu.VMEM((CH, D), f32),
        pltpu.VMEM((CH,), f32),
        pltpu.VMEM((CH,), f32),
        pltpu.VMEM((CH,), f32),
        pltpu.VMEM((CH,), f32),
        pltpu.SemaphoreType.DMA,
    ]
    return pl.kernel(
        _sc_body, out_type=out_type, mesh=mesh, scratch_types=scratch,
        compiler_params=pltpu.CompilerParams(
            needs_layout_passes=False, use_tc_tiling_on_sc=False),
    )(users, pos, negf, Gu, Gi, bu, bi, nmatf, cmatf)


def _softplus(x):
    return jnp.maximum(x, 0.0) + jnp.log1p(jnp.exp(-jnp.abs(x)))


def _tc_body(gu, gi, ps, pw, ns, nw, ss, sim, out, accs):
    i = pl.program_id(0)

    @pl.when(i == 0)
    def _init():
        accs[0] = jnp.sum(pw[...] * _softplus(-ps[...])) \
            + LM * jnp.sum(sim[...] * _softplus(-ss[...]))
        accs[1] = 0.0

    accs[0] += (NEG_WEIGHT / K) * jnp.sum(nw[...] * _softplus(ns[...]))
    accs[1] += jnp.sum(gu[...] * gu[...]) + jnp.sum(gi[...] * gi[...])

    @pl.when(i == TC_GRID - 1)
    def _fini():
        out[...] = jnp.reshape(accs[0] + (GAMMA * 0.5) * accs[1], (1, 1))


@jax.jit
def _tc_call(GuR, GiR, ps, pw, ns, nw, ss, sim):
    grid = (TC_GRID,)
    specs = [
        pl.BlockSpec((TBL_BLK, 128), lambda i: (i, 0)),
        pl.BlockSpec((TBL_BLK, 128), lambda i: (i, 0)),
        pl.BlockSpec((32, 128), lambda i: (0, 0)),
        pl.BlockSpec((32, 128), lambda i: (0, 0)),
        pl.BlockSpec((1, 128, 128), lambda i: (i, 0, 0)),
        pl.BlockSpec((1, 128, 128), lambda i: (i, 0, 0)),
        pl.BlockSpec((320, 128), lambda i: (0, 0)),
        pl.BlockSpec((320, 128), lambda i: (0, 0)),
    ]
    return pl.pallas_call(
        _tc_body,
        grid=grid,
        in_specs=specs,
        out_specs=pl.BlockSpec((1, 1), lambda i: (0, 0)),
        out_shape=jax.ShapeDtypeStruct((1, 1), jnp.float32),
        scratch_shapes=[pltpu.SMEM((2,), jnp.float32)],
    )(GuR, GiR, ps, pw, ns, nw, ss, sim)


def kernel(users, pos_items, neg_items, Gu, Gi, beta_uD, beta_iD,
           ii_neighbor_mat, ii_constraint_mat):
    users = users.astype(jnp.int32)
    pos = pos_items.astype(jnp.int32)
    negf = neg_items.reshape(-1).astype(jnp.int32)
    nmatf = ii_neighbor_mat.reshape(-1).astype(jnp.int32)
    cmatf = ii_constraint_mat.reshape(-1)

    ps, pw, nsc, nwt, ssc, sim = _sc_call(
        users, pos, negf, Gu, Gi, beta_uD, beta_iD, nmatf, cmatf)

    out = _tc_call(
        Gu.reshape(TBL_ROWS, 128),
        Gi.reshape(TBL_ROWS, 128),
        ps.reshape(32, 128),
        pw.reshape(32, 128),
        nsc.reshape(TC_GRID, 128, 128),
        nwt.reshape(TC_GRID, 128, 128),
        ssc.reshape(320, 128),
        sim.reshape(320, 128),
    )
    return out[0, 0]


# speedup vs baseline: 1.0850x; 1.0850x over previous
import jax
import jax.numpy as jnp
from jax import lax
from jax.experimental import pallas as pl
from jax.experimental.pallas import tpu as pltpu
from jax.experimental.pallas import tpu_sc as plsc

B = 4096
K = 200
D = 32
NN = 10
W1 = 1e-07
W2 = 1.0
W3 = 1e-07
W4 = 1.0
NEG_WEIGHT = 200.0
GAMMA = 1e-04
LM = 2.75

NC = 2
NS = 16
L = 16
NW = NC * NS
BPW = B // NW
CH = 128
NBUF = 4

NEG_CHUNKS = BPW * K // CH
NBH_CHUNKS = BPW * NN // CH
TBL_ROWS = 250000
TC_GRID = 50
TBL_BLK = TBL_ROWS // TC_GRID


def _iota16():
    return lax.iota(jnp.int32, L)


def _sc_body(users, pos, negf, Gu, Gi, bu, bi, nmatf, cmatf,
             pos_s_o, pos_w_o, neg_s_o, neg_w_o, sc_s_o, sim_o,
             uidx_v, pidx_v, urows_v, prows_v, buv, bipv,
             negblk_v, c0, c1, c2, c3, g0, g1, g2, g3, b0, b1, b2, b3,
             tmpS, tmpW, sfull_v, wfull_v,
             semA, semB, semC, semD, semN, semU, semP, semBU, semBI,
             semS1, semS2):
    wid = lax.axis_index("s") * NC + lax.axis_index("c")
    base = pl.multiple_of(wid * BPW, BPW)
    cbufs = (c0, c1, c2, c3)
    gbufs = (g0, g1, g2, g3)
    bbufs = (b0, b1, b2, b3)
    sems = (semA, semB, semC, semD)

    pltpu.sync_copy(users.at[pl.ds(base, BPW)], uidx_v)
    pltpu.sync_copy(pos.at[pl.ds(base, BPW)], pidx_v)
    d0 = pltpu.async_copy(
        negf.at[pl.ds(pl.multiple_of(base * K, 8), BPW * K)], negblk_v, semN)

    dms = [
        pltpu.async_copy(Gu.at[uidx_v], urows_v, semU),
        pltpu.async_copy(Gi.at[pidx_v], prows_v, semP),
        pltpu.async_copy(bu.at[uidx_v], buv.at[pl.ds(0, BPW)], semBU),
        pltpu.async_copy(bi.at[pidx_v], bipv, semBI),
    ]
    for dm in dms:
        dm.wait()

    def pos_group(g, _):
        jv = g * L + _iota16()
        acc = jnp.zeros((L,), jnp.float32)
        for d in range(D):
            dv = jnp.full((L,), d, jnp.int32)
            acc = acc + (plsc.load_gather(prows_v, [jv, dv])
                         * plsc.load_gather(urows_v, [jv, dv]))
        tmpS[pl.ds(g * L, L)] = acc
        w = W1 + W2 * buv[pl.ds(g * L, L)] * bipv[pl.ds(g * L, L)]
        tmpW[pl.ds(g * L, L)] = w
        return 0
    lax.fori_loop(0, BPW // L, pos_group, 0)
    pltpu.sync_copy(tmpS, pos_s_o.at[pl.ds(base, BPW)])
    pltpu.sync_copy(tmpW, pos_w_o.at[pl.ds(base, BPW)])

    def nbh_chunk(t, _):
        jbase = t * CH

        def build_idx(g, _):
            jv = jbase + g * L + _iota16()
            ev = lax.div(jv, NN)
            rv = lax.rem(jv, NN)
            pid = plsc.load_gather(pidx_v, [ev])
            c0[pl.ds(g * L, L)] = pid * NN + rv
            return 0
        lax.fori_loop(0, CH // L, build_idx, 0)

        dn = pltpu.async_copy(nmatf.at[c0], c1, semS1)
        dsim = pltpu.async_copy(cmatf.at[c0], tmpW, semS2)
        dn.wait()
        pltpu.async_copy(Gi.at[c1], g0, semS1).wait()

        def dot_group(g, _):
            jloc = g * L + _iota16()
            ev = lax.div(jbase + jloc, NN)
            acc = jnp.zeros((L,), jnp.float32)
            for d in range(D):
                dv = jnp.full((L,), d, jnp.int32)
                acc = acc + (plsc.load_gather(g0, [jloc, dv])
                             * plsc.load_gather(urows_v, [ev, dv]))
            tmpS[pl.ds(g * L, L)] = acc
            return 0
        lax.fori_loop(0, CH // L, dot_group, 0)

        dsim.wait()
        off = pl.multiple_of(base * NN + jbase, 8)
        pltpu.sync_copy(tmpS, sc_s_o.at[pl.ds(off, CH)])
        pltpu.sync_copy(tmpW, sim_o.at[pl.ds(off, CH)])
        return 0
    lax.fori_loop(0, NBH_CHUNKS, nbh_chunk, 0)

    d0.wait()

    def fire_neg(t, cbuf, gbuf, bbuf, sem):
        def bld(g, _):
            jv = t * CH + g * L + _iota16()
            cbuf[pl.ds(g * L, L)] = plsc.load_gather(negblk_v, [jv])
            return 0
        lax.fori_loop(0, CH // L, bld, 0)
        return (pltpu.async_copy(Gi.at[cbuf], gbuf, sem),
                pltpu.async_copy(bi.at[cbuf], bbuf, sem))

    def compute_neg(t, gbuf, bbuf):
        e0 = lax.div(t * CH, K)
        e1 = jnp.minimum(e0 + 1, BPW - 1)
        cut = (e0 + 1) * K - t * CH
        u0h = (urows_v[e0, pl.ds(0, L)], urows_v[e0, pl.ds(L, L)])
        u1h = (urows_v[e1, pl.ds(0, L)], urows_v[e1, pl.ds(L, L)])
        u0s = [u0h[d // L][d % L] for d in range(D)]
        u1s = [u1h[d // L][d % L] for d in range(D)]
        bu0 = buv[pl.ds(e0, L)][0]
        bu1 = buv[pl.ds(e1, L)][0]

        def group(g, _):
            jv = g * L + _iota16()
            m = jv < cut
            acc = jnp.zeros((L,), jnp.float32)
            for d in range(D):
                dv = jnp.full((L,), d, jnp.int32)
                ub = jnp.where(m, u0s[d], u1s[d])
                acc = acc + plsc.load_gather(gbuf, [jv, dv]) * ub
            sfull_v[pl.ds(t * CH + g * L, L)] = acc
            busel = jnp.where(m, bu0, bu1)
            wfull_v[pl.ds(t * CH + g * L, L)] = \
                W3 + W4 * busel * bbuf[pl.ds(g * L, L)]
            return 0
        lax.fori_loop(0, CH // L, group, 0)

    def neg_quad(q, _):
        t0 = NBUF * q
        ds = [fire_neg(t0 + p, cbufs[p], gbufs[p], bbufs[p], sems[p])
              for p in range(NBUF)]
        for p in range(NBUF):
            ds[p][0].wait()
            ds[p][1].wait()
            compute_neg(t0 + p, gbufs[p], bbufs[p])
        return 0
    lax.fori_loop(0, NEG_CHUNKS // NBUF, neg_quad, 0)

    off = pl.multiple_of(base * K, 8)
    pltpu.sync_copy(sfull_v, neg_s_o.at[pl.ds(off, BPW * K)])
    pltpu.sync_copy(wfull_v, neg_w_o.at[pl.ds(off, BPW * K)])


@jax.jit
def _sc_call(users, pos, negf, Gu, Gi, bu, bi, nmatf, cmatf):
    mesh = plsc.VectorSubcoreMesh(core_axis_name="c", subcore_axis_name="s")
    f32 = jnp.float32
    i32 = jnp.int32
    out_type = (
        jax.ShapeDtypeStruct((B,), f32),
        jax.ShapeDtypeStruct((B,), f32),
        jax.ShapeDtypeStruct((B * K,), f32),
        jax.ShapeDtypeStruct((B * K,), f32),
        jax.ShapeDtypeStruct((B * NN,), f32),
        jax.ShapeDtypeStruct((B * NN,), f32),
    )
    scratch = [
        pltpu.VMEM((BPW,), i32),
        pltpu.VMEM((BPW,), i32),
        pltpu.VMEM((BPW, D), f32),
        pltpu.VMEM((BPW, D), f32),
        pltpu.VMEM((BPW + L,), f32),
        pltpu.VMEM((BPW,), f32),
        pltpu.VMEM((BPW * K,), i32),
        pltpu.VMEM((CH,), i32),
        pltpu.VMEM((CH,), i32),
        pltpu.VMEM((CH,), i32),
        pltpu.VMEM((CH,), i32),
        pltpu.VMEM((CH, D), f32),
        pltpu.VMEM((CH, D), f32),
        pltpu.VMEM((CH, D), f32),
        pltpu.VMEM((CH, D), f32),
        pltpu.VMEM((CH,), f32),
        pltpu.VMEM((CH,), f32),
        pltpu.VMEM((CH,), f32),
        pltpu.VMEM((CH,), f32),
        pltpu.VMEM((CH,), f32),
        pltpu.VMEM((CH,), f32),
        pltpu.VMEM((BPW * K,), f32),
        pltpu.VMEM((BPW * K,), f32),
        pltpu.SemaphoreType.DMA,
        pltpu.SemaphoreType.DMA,
        pltpu.SemaphoreType.DMA,
        pltpu.SemaphoreType.DMA,
        pltpu.SemaphoreType.DMA,
        pltpu.SemaphoreType.DMA,
        pltpu.SemaphoreType.DMA,
        pltpu.SemaphoreType.DMA,
        pltpu.SemaphoreType.DMA,
        pltpu.SemaphoreType.DMA,
        pltpu.SemaphoreType.DMA,
    ]
    return pl.kernel(
        _sc_body, out_type=out_type, mesh=mesh, scratch_types=scratch,
        compiler_params=pltpu.CompilerParams(
            needs_layout_passes=False, use_tc_tiling_on_sc=False),
    )(users, pos, negf, Gu, Gi, bu, bi, nmatf, cmatf)


def _softplus(x):
    return jnp.maximum(x, 0.0) + jnp.log1p(jnp.exp(-jnp.abs(x)))


def _tc_body(gu, gi, ps, pw, ns, nw, ss, sim, out, accs):
    i = pl.program_id(0)

    @pl.when(i == 0)
    def _init():
        accs[0] = jnp.sum(pw[...] * _softplus(-ps[...])) \
            + LM * jnp.sum(sim[...] * _softplus(-ss[...]))
        accs[1] = 0.0

    accs[0] += (NEG_WEIGHT / K) * jnp.sum(nw[...] * _softplus(ns[...]))
    accs[1] += jnp.sum(gu[...] * gu[...]) + jnp.sum(gi[...] * gi[...])

    @pl.when(i == TC_GRID - 1)
    def _fini():
        out[...] = jnp.reshape(accs[0] + (GAMMA * 0.5) * accs[1], (1, 1))


@jax.jit
def _tc_call(GuR, GiR, ps, pw, ns, nw, ss, sim):
    grid = (TC_GRID,)
    specs = [
        pl.BlockSpec((TBL_BLK, 128), lambda i: (i, 0)),
        pl.BlockSpec((TBL_BLK, 128), lambda i: (i, 0)),
        pl.BlockSpec((32, 128), lambda i: (0, 0)),
        pl.BlockSpec((32, 128), lambda i: (0, 0)),
        pl.BlockSpec((1, 128, 128), lambda i: (i, 0, 0)),
        pl.BlockSpec((1, 128, 128), lambda i: (i, 0, 0)),
        pl.BlockSpec((320, 128), lambda i: (0, 0)),
        pl.BlockSpec((320, 128), lambda i: (0, 0)),
    ]
    return pl.pallas_call(
        _tc_body,
        grid=grid,
        in_specs=specs,
        out_specs=pl.BlockSpec((1, 1), lambda i: (0, 0)),
        out_shape=jax.ShapeDtypeStruct((1, 1), jnp.float32),
        scratch_shapes=[pltpu.SMEM((2,), jnp.float32)],
    )(GuR, GiR, ps, pw, ns, nw, ss, sim)


def kernel(users, pos_items, neg_items, Gu, Gi, beta_uD, beta_iD,
           ii_neighbor_mat, ii_constraint_mat):
    users = users.astype(jnp.int32)
    pos = pos_items.astype(jnp.int32)
    negf = neg_items.reshape(-1).astype(jnp.int32)
    nmatf = ii_neighbor_mat.reshape(-1).astype(jnp.int32)
    cmatf = ii_constraint_mat.reshape(-1)

    ps, pw, nsc, nwt, ssc, sim = _sc_call(
        users, pos, negf, Gu, Gi, beta_uD, beta_iD, nmatf, cmatf)

    out = _tc_call(
        Gu.reshape(TBL_ROWS, 128),
        Gi.reshape(TBL_ROWS, 128),
        ps.reshape(32, 128),
        pw.reshape(32, 128),
        nsc.reshape(TC_GRID, 128, 128),
        nwt.reshape(TC_GRID, 128, 128),
        ssc.reshape(320, 128),
        sim.reshape(320, 128),
    )
    return out[0, 0]
